# baseline (speedup 1.0000x reference)
import functools

import jax
import jax.numpy as jnp
from jax import lax
from jax.experimental import pallas as pl
from jax.experimental.pallas import tpu as pltpu
from jax.experimental.pallas import tpu_sc as plsc

N = 10000
E = 320000
DE = 16
HID = 128
DEC = 64

NC = 2
NS = 16
NW = NC * NS
CHUNK = 128
CPW = 80
EPW = CHUNK * CPW
E_PAD = NW * EPW
FAST = 0
CPW_F = 80
CPW_S = 80
TOTCH = E_PAD // CHUNK
N_PAD = 10240
RPT = N_PAD // NS
CW = 16

f32 = jnp.float32
bf16 = jnp.bfloat16
i32 = jnp.int32

_mesh = lambda: plsc.VectorSubcoreMesh(core_axis_name="c", subcore_axis_name="s")

u16 = jnp.uint16
u32 = jnp.uint32


def _pack(a, half):
    lo = lax.bitcast_convert_type(a[:, :half].astype(bf16), u16).astype(u32)
    hi = lax.bitcast_convert_type(a[:, half:].astype(bf16), u16).astype(u32)
    return lax.bitcast_convert_type(lo | (hi << 16), i32)


def _unpack(g, half):
    u = lax.bitcast_convert_type(g, u32)
    lo = lax.bitcast_convert_type((u & 0xFFFF).astype(u16), bf16).astype(f32)
    hi = lax.bitcast_convert_type((u >> 16).astype(u16), bf16).astype(f32)
    return jnp.concatenate([lo, hi], axis=1)


def _make_gather(D):
    dt = i32

    @functools.partial(
        pl.kernel,
        out_type=(
            jax.ShapeDtypeStruct((E_PAD, D), dt),
            jax.ShapeDtypeStruct((E_PAD, D), dt),
        ),
        compiler_params=pltpu.CompilerParams(use_tc_tiling_on_sc=False),
        mesh=_mesh(),
        scratch_types=[
            pltpu.VMEM((CPW_F, CHUNK), i32),
            pltpu.VMEM((CPW_F, CHUNK), i32),
            pltpu.VMEM((2, CHUNK, D), dt),
            pltpu.VMEM((2, CHUNK, D), dt),
            pltpu.SemaphoreType.DMA,
            pltpu.SemaphoreType.DMA,
            pltpu.SemaphoreType.DMA,
            pltpu.SemaphoreType.DMA,
            pltpu.SemaphoreType.DMA,
            pltpu.SemaphoreType.DMA,
            pltpu.SemaphoreType.DMA,
            pltpu.SemaphoreType.DMA,
        ],
    )
    def gather(t0, t1, i0, i1, o0, o1, i0_v, i1_v, b0, b1,
               g0a, g1a, g0b, g1b, w0a, w1a, w0b, w1b):
        c = lax.axis_index("c")
        t = lax.axis_index("s")
        is_fast = c == FAST
        coff = jnp.where(is_fast, t * CPW_F, NS * CPW_F + t * CPW_S)
        npair = jnp.where(is_fast, CPW_F // 2, CPW_S // 2)

        @pl.when(is_fast)
        def _():
            pltpu.sync_copy(i0.at[pl.ds(t * CPW_F, CPW_F)], i0_v)
            pltpu.sync_copy(i1.at[pl.ds(t * CPW_F, CPW_F)], i1_v)

        @pl.when(jnp.logical_not(is_fast))
        def _():
            s0 = NS * CPW_F + t * CPW_S
            pltpu.sync_copy(i0.at[pl.ds(s0, CPW_S)], i0_v.at[pl.ds(0, CPW_S)])
            pltpu.sync_copy(i1.at[pl.ds(s0, CPW_S)], i1_v.at[pl.ds(0, CPW_S)])

        base = coff * CHUNK
        gsem = ((g0a, g1a), (g0b, g1b))
        wsem = ((w0a, w1a), (w0b, w1b))
        bufs = ((b0.at[0], b1.at[0]), (b0.at[1], b1.at[1]))

        HC = CHUNK // 2

        def fire_g(j, k):
            pltpu.async_copy(
                t0.at[i0_v.at[j, pl.ds(0, HC)]], bufs[k][0].at[pl.ds(0, HC)], gsem[k][0])
            pltpu.async_copy(
                t0.at[i0_v.at[j, pl.ds(HC, HC)]], bufs[k][0].at[pl.ds(HC, HC)], gsem[k][0])
            pltpu.async_copy(
                t1.at[i1_v.at[j, pl.ds(0, HC)]], bufs[k][1].at[pl.ds(0, HC)], gsem[k][1])
            pltpu.async_copy(
                t1.at[i1_v.at[j, pl.ds(HC, HC)]], bufs[k][1].at[pl.ds(HC, HC)], gsem[k][1])

        def wait_g(j, k):
            pltpu.make_async_copy(t0.at[i0_v.at[j]], bufs[k][0], gsem[k][0]).wait()
            pltpu.make_async_copy(t1.at[i1_v.at[j]], bufs[k][1], gsem[k][1]).wait()

        def fire_w(j, k):
            d = pl.ds(base + j * CHUNK, CHUNK)
            pltpu.async_copy(bufs[k][0], o0.at[d], wsem[k][0])
            pltpu.async_copy(bufs[k][1], o1.at[d], wsem[k][1])

        def wait_w(j, k):
            d = pl.ds(base + j * CHUNK, CHUNK)
            pltpu.make_async_copy(bufs[k][0], o0.at[d], wsem[k][0]).wait()
            pltpu.make_async_copy(bufs[k][1], o1.at[d], wsem[k][1]).wait()

        fire_g(0, 0)

        def body(jj, carry):
            j0 = 2 * jj
            j1 = j0 + 1

            @pl.when(jj > 0)
            def _():
                wait_w(j1 - 2, 1)

            fire_g(j1, 1)
            wait_g(j0, 0)
            fire_w(j0, 0)
            wait_g(j1, 1)
            wait_w(j0, 0)

            @pl.when(jj + 1 < npair)
            def _():
                fire_g(j0 + 2, 0)

            fire_w(j1, 1)
            return carry

        lax.fori_loop(0, npair, body, 0)
        wait_w(2 * npair - 1, 1)

    return gather


def _make_scatter():
    @functools.partial(
        pl.kernel,
        out_type=(
            jax.ShapeDtypeStruct((N_PAD, HID), f32),
            jax.ShapeDtypeStruct((N_PAD, HID), f32),
        ),
        mesh=_mesh(),
        scratch_types=[
            pltpu.VMEM((CPW, CHUNK), i32),
            pltpu.VMEM((2, CHUNK, HID), f32),
            pltpu.VMEM_SHARED((N_PAD, HID), f32),
            pltpu.SemaphoreType.DMA,
            pltpu.SemaphoreType.DMA,
        ],
    )
    def scatter(m, idx, zeros, o0, o1, i_v, buf, acc, la, lb):
        c = lax.axis_index("c")
        t = lax.axis_index("s")
        wid = t * NC + c
        pltpu.sync_copy(zeros, buf.at[0])

        def zbody(k, carry):
            pltpu.sync_copy(buf.at[0], acc.at[pl.ds(t * RPT + k * CHUNK, CHUNK)])
            return carry

        lax.fori_loop(0, RPT // CHUNK, zbody, 0)
        plsc.subcore_barrier()

        pltpu.sync_copy(idx.at[wid], i_v)
        base = wid * EPW
        sems = (la, lb)

        def fire_l(j, k):
            pltpu.async_copy(m.at[pl.ds(base + j * CHUNK, CHUNK)], buf.at[k], sems[k])

        def wait_l(j, k):
            pltpu.make_async_copy(
                m.at[pl.ds(base + j * CHUNK, CHUNK)], buf.at[k], sems[k]).wait()

        fire_l(0, 0)

        def body(jj, carry):
            j0 = 2 * jj
            j1 = j0 + 1
            fire_l(j1, 1)
            wait_l(j0, 0)
            pltpu.sync_copy(buf.at[0], acc.at[i_v.at[j0]], add=True)

            @pl.when(jj + 1 < CPW // 2)
            def _():
                fire_l(j0 + 2, 0)

            wait_l(j1, 1)
            pltpu.sync_copy(buf.at[1], acc.at[i_v.at[j1]], add=True)
            return carry

        lax.fori_loop(0, CPW // 2, body, 0)
        plsc.subcore_barrier()

        def wbody(k, carry):
            r = t * RPT + k * CHUNK
            pltpu.sync_copy(acc.at[pl.ds(r, CHUNK)], buf.at[0])

            @pl.when(c == 0)
            def _():
                pltpu.sync_copy(buf.at[0], o0.at[pl.ds(r, CHUNK)])

            @pl.when(c == 1)
            def _():
                pltpu.sync_copy(buf.at[0], o1.at[pl.ds(r, CHUNK)])

            return carry

        lax.fori_loop(0, RPT // CHUNK, wbody, 0)

    return scatter


def _make_counts():
    @functools.partial(
        pl.kernel,
        out_type=(
            jax.ShapeDtypeStruct((N_PAD, CW), f32),
            jax.ShapeDtypeStruct((N_PAD, CW), f32),
        ),
        compiler_params=pltpu.CompilerParams(use_tc_tiling_on_sc=False),
        mesh=_mesh(),
        scratch_types=[
            pltpu.VMEM((CPW, CHUNK), i32),
            pltpu.VMEM((CHUNK, CW), f32),
            pltpu.VMEM((CHUNK, CW), f32),
            pltpu.VMEM_SHARED((N_PAD, CW), f32),
            pltpu.SemaphoreType.DMA,
        ],
    )
    def counts(idx, zeros, ones, o0, o1, i_v, zbuf, obuf, acc, sem):
        c = lax.axis_index("c")
        t = lax.axis_index("s")
        wid = t * NC + c
        pltpu.sync_copy(zeros, zbuf)
        pltpu.sync_copy(ones, obuf)

        def zbody(k, carry):
            pltpu.sync_copy(zbuf, acc.at[pl.ds(t * RPT + k * CHUNK, CHUNK)])
            return carry

        lax.fori_loop(0, RPT // CHUNK, zbody, 0)
        plsc.subcore_barrier()

        pltpu.sync_copy(idx.at[wid], i_v)

        def body(j, carry):
            pltpu.sync_copy(obuf, acc.at[i_v.at[j]], add=True)
            return carry

        lax.fori_loop(0, CPW, body, 0)
        plsc.subcore_barrier()

        def wbody(k, carry):
            r = t * RPT + k * CHUNK
            pltpu.sync_copy(acc.at[pl.ds(r, CHUNK)], zbuf)

            @pl.when(c == 0)
            def _():
                pltpu.sync_copy(zbuf, o0.at[pl.ds(r, CHUNK)])

            @pl.when(c == 1)
            def _():
                pltpu.sync_copy(zbuf, o1.at[pl.ds(r, CHUNK)])

            return carry

        lax.fori_loop(0, RPT // CHUNK, wbody, 0)

    return counts


_BR_E = 640
_BR_N = 1000


def _premul(x, wa, wb, dout):
    def body(x_r, wa_r, wb_r, a_r, b_r):
        xv = x_r[:]
        a_r[:] = _pack(jnp.dot(xv, wa_r[:], preferred_element_type=f32), dout // 2)
        b_r[:] = _pack(jnp.dot(xv, wb_r[:], preferred_element_type=f32), dout // 2)

    return pl.pallas_call(
        body,
        grid=(N // _BR_N,),
        in_specs=[
            pl.BlockSpec((_BR_N, HID), lambda i: (i, 0)),
            pl.BlockSpec((HID, dout), lambda i: (0, 0)),
            pl.BlockSpec((HID, dout), lambda i: (0, 0)),
        ],
        out_specs=[
            pl.BlockSpec((_BR_N, dout // 2), lambda i: (i, 0)),
            pl.BlockSpec((_BR_N, dout // 2), lambda i: (i, 0)),
        ],
        out_shape=[
            jax.ShapeDtypeStruct((N, dout // 2), i32),
            jax.ShapeDtypeStruct((N, dout // 2), i32),
        ],
    )(x, wa, wb)


def _edge_mlp(g0, g1, ea, wc, b1):
    nea = ea.shape[0] // _BR_E - 1

    def body(g0_r, g1_r, e_r, c_r, b1_r, o_r):
        z = (_unpack(g0_r[:], HID // 2) + _unpack(g1_r[:], HID // 2)
             + jnp.dot(e_r[:], c_r[:], preferred_element_type=f32) + b1_r[:])
        o_r[:] = jnp.maximum(z, 0.0)

    return pl.pallas_call(
        body,
        grid=(E_PAD // _BR_E,),
        in_specs=[
            pl.BlockSpec((_BR_E, HID // 2), lambda i: (i, 0)),
            pl.BlockSpec((_BR_E, HID // 2), lambda i: (i, 0)),
            pl.BlockSpec((_BR_E, DE), lambda i: (jnp.minimum(i, nea), 0)),
            pl.BlockSpec((DE, HID), lambda i: (0, 0)),
            pl.BlockSpec((1, HID), lambda i: (0, 0)),
        ],
        out_specs=pl.BlockSpec((_BR_E, HID), lambda i: (i, 0)),
        out_shape=jax.ShapeDtypeStruct((E_PAD, HID), f32),
    )(g0, g1, ea, wc, b1)


def _node_mlp(x, s0, s1, c0, c1, mw2, mb2, wx, wm, b1, w2, b2, wa, wb, dout):
    def body(x_r, s0_r, s1_r, c0_r, c1_r, mw2_r, mb2_r, wx_r, wm_r, b1_r, w2_r, b2_r,
             wa_r, wb_r, h_r, a_r, b_r):
        craw = c0_r[:, 0:1] + c1_r[:, 0:1]
        cnt = jnp.maximum(craw, 1.0)
        rz = (s0_r[:] + s1_r[:]) / cnt
        mean = (jnp.dot(rz, mw2_r[:], preferred_element_type=f32) + mb2_r[:]
                ) * (craw > 0.0)
        u = jnp.maximum(
            jnp.dot(x_r[:], wx_r[:], preferred_element_type=f32)
            + jnp.dot(mean, wm_r[:], preferred_element_type=f32) + b1_r[:], 0.0)
        h = jnp.maximum(
            jnp.dot(u, w2_r[:], preferred_element_type=f32) + b2_r[:], 0.0)
        h_r[:] = h
        a_r[:] = _pack(jnp.dot(h, wa_r[:], preferred_element_type=f32), dout // 2)
        b_r[:] = _pack(jnp.dot(h, wb_r[:], preferred_element_type=f32), dout // 2)

    return pl.pallas_call(
        body,
        grid=(N // _BR_N,),
        in_specs=[
            pl.BlockSpec((_BR_N, HID), lambda i: (i, 0)),
            pl.BlockSpec((_BR_N, HID), lambda i: (i, 0)),
            pl.BlockSpec((_BR_N, HID), lambda i: (i, 0)),
            pl.BlockSpec((_BR_N, CW), lambda i: (i, 0)),
            pl.BlockSpec((_BR_N, CW), lambda i: (i, 0)),
            pl.BlockSpec((HID, HID), lambda i: (0, 0)),
            pl.BlockSpec((1, HID), lambda i: (0, 0)),
            pl.BlockSpec((HID, HID), lambda i: (0, 0)),
            pl.BlockSpec((HID, HID), lambda i: (0, 0)),
            pl.BlockSpec((1, HID), lambda i: (0, 0)),
            pl.BlockSpec((HID, HID), lambda i: (0, 0)),
            pl.BlockSpec((1, HID), lambda i: (0, 0)),
            pl.BlockSpec((HID, dout), lambda i: (0, 0)),
            pl.BlockSpec((HID, dout), lambda i: (0, 0)),
        ],
        out_specs=[
            pl.BlockSpec((_BR_N, HID), lambda i: (i, 0)),
            pl.BlockSpec((_BR_N, dout // 2), lambda i: (i, 0)),
            pl.BlockSpec((_BR_N, dout // 2), lambda i: (i, 0)),
        ],
        out_shape=[
            jax.ShapeDtypeStruct((N, HID), f32),
            jax.ShapeDtypeStruct((N, dout // 2), i32),
            jax.ShapeDtypeStruct((N, dout // 2), i32),
        ],
    )(x, s0, s1, c0, c1, mw2, mb2, wx, wm, b1, w2, b2, wa, wb)


def _decoder_mlp(g0, g1, ea, wc, b1, w2, b2, w3, b3):
    nea = ea.shape[0] // _BR_E - 1

    def body(g0_r, g1_r, e_r, c_r, b1_r, w2_r, b2_r, w3_r, b3_r, o_r):
        z = jnp.maximum(
            _unpack(g0_r[:], DEC // 2) + _unpack(g1_r[:], DEC // 2)
            + jnp.dot(e_r[:], c_r[:], preferred_element_type=f32) + b1_r[:], 0.0)
        d = jnp.maximum(jnp.dot(z, w2_r[:], preferred_element_type=f32) + b2_r[:], 0.0)
        o_r[:] = jnp.dot(d, w3_r[:], preferred_element_type=f32) + b3_r[0, 0]

    return pl.pallas_call(
        body,
        grid=(E_PAD // _BR_E,),
        in_specs=[
            pl.BlockSpec((_BR_E, DEC // 2), lambda i: (i, 0)),
            pl.BlockSpec((_BR_E, DEC // 2), lambda i: (i, 0)),
            pl.BlockSpec((_BR_E, DE), lambda i: (jnp.minimum(i, nea), 0)),
            pl.BlockSpec((DE, DEC), lambda i: (0, 0)),
            pl.BlockSpec((1, DEC), lambda i: (0, 0)),
            pl.BlockSpec((DEC, DEC // 2), lambda i: (0, 0)),
            pl.BlockSpec((1, DEC // 2), lambda i: (0, 0)),
            pl.BlockSpec((DEC // 2, 1), lambda i: (0, 0)),
            pl.BlockSpec((1, 1), lambda i: (0, 0)),
        ],
        out_specs=pl.BlockSpec((_BR_E, 1), lambda i: (i, 0)),
        out_shape=jax.ShapeDtypeStruct((E_PAD, 1), f32),
    )(g0, g1, ea, wc, b1, w2, b2, w3, b3)


def kernel(x, edge_index, edge_attr, params):
    src = edge_index[0]
    dst = edge_index[1]
    pad0 = jnp.zeros((E_PAD - E,), i32)
    padn = jnp.full((E_PAD - E,), N, i32)
    dst_g = jnp.concatenate([dst, pad0]).reshape(TOTCH, CHUNK)
    src_g = jnp.concatenate([src, pad0]).reshape(TOTCH, CHUNK)
    dst_s = jnp.concatenate([dst, padn]).reshape(NW, CPW, CHUNK)
    zeros_h = jnp.zeros((CHUNK, HID), f32)
    zeros_c = jnp.zeros((CHUNK, CW), f32)
    ones_c = jnp.ones((CHUNK, CW), f32)

    gather_h = _make_gather(HID // 2)
    gather_d = _make_gather(DEC // 2)
    scatter = _make_scatter()
    counts = _make_counts()

    c0, c1 = counts(dst_s, zeros_c, ones_c)

    w1 = params['l0_msg_W1']
    xa, xb = _premul(x, w1[:HID], w1[HID:2 * HID], HID)
    h = x
    for l in range(3):
        w1 = params['l%d_msg_W1' % l]
        g0, g1 = gather_h(xa, xb, dst_g, src_g)
        rz = _edge_mlp(
            g0, g1, edge_attr, w1[2 * HID:],
            params['l%d_msg_b1' % l].reshape(1, HID))
        s0, s1 = scatter(rz, dst_s, zeros_h)
        nw1 = params['l%d_node_W1' % l]
        if l < 2:
            nxt = params['l%d_msg_W1' % (l + 1)]
            wa, wb, dout = nxt[:HID], nxt[HID:2 * HID], HID
        else:
            dw1 = params['dec_W1']
            wa, wb, dout = dw1[:HID], dw1[HID:2 * HID], DEC
        h, xa, xb = _node_mlp(
            h, s0, s1, c0, c1,
            params['l%d_msg_W2' % l],
            params['l%d_msg_b2' % l].reshape(1, HID),
            nw1[:HID], nw1[HID:],
            params['l%d_node_b1' % l].reshape(1, HID),
            params['l%d_node_W2' % l],
            params['l%d_node_b2' % l].reshape(1, HID),
            wa, wb, dout)

    gd0, gd1 = gather_d(xa, xb, src_g, dst_g)
    out = _decoder_mlp(
        gd0, gd1, edge_attr, params['dec_W1'][2 * HID:],
        params['dec_b1'].reshape(1, DEC),
        params['dec_W2'],
        params['dec_b2'].reshape(1, DEC // 2),
        params['dec_W3'],
        params['dec_b3'].reshape(1, 1))
    return out[:E, 0]


# speedup vs baseline: 1.1250x; 1.1250x over previous
import functools

import jax
import jax.numpy as jnp
from jax import lax
from jax.experimental import pallas as pl
from jax.experimental.pallas import tpu as pltpu
from jax.experimental.pallas import tpu_sc as plsc

N = 10000
E = 320000
DE = 16
HID = 128
DEC = 64

NC = 2
NS = 16
NW = NC * NS
CHUNK = 128
CPW = 80
EPW = CHUNK * CPW
E_PAD = NW * EPW
FAST = 0
CPW_F = 80
CPW_S = 80
TOTCH = E_PAD // CHUNK
N_PAD = 10240
RPT = N_PAD // NS
CW = 16

f32 = jnp.float32
bf16 = jnp.bfloat16
i32 = jnp.int32

_mesh = lambda: plsc.VectorSubcoreMesh(core_axis_name="c", subcore_axis_name="s")

u16 = jnp.uint16
u32 = jnp.uint32


def _pack(a, half):
    lo = lax.bitcast_convert_type(a[:, :half].astype(bf16), u16).astype(u32)
    hi = lax.bitcast_convert_type(a[:, half:].astype(bf16), u16).astype(u32)
    return lax.bitcast_convert_type(lo | (hi << 16), i32)


def _unpack(g, half):
    u = lax.bitcast_convert_type(g, u32)
    lo = lax.bitcast_convert_type((u & 0xFFFF).astype(u16), bf16).astype(f32)
    hi = lax.bitcast_convert_type((u >> 16).astype(u16), bf16).astype(f32)
    return jnp.concatenate([lo, hi], axis=1)


def _make_gather(D):
    dt = i32

    @functools.partial(
        pl.kernel,
        out_type=(
            jax.ShapeDtypeStruct((E_PAD, D), dt),
            jax.ShapeDtypeStruct((E_PAD, D), dt),
        ),
        compiler_params=pltpu.CompilerParams(use_tc_tiling_on_sc=False),
        mesh=_mesh(),
        scratch_types=[
            pltpu.VMEM((CPW_F, CHUNK), i32),
            pltpu.VMEM((CPW_F, CHUNK), i32),
            pltpu.VMEM((2, CHUNK, D), dt),
            pltpu.VMEM((2, CHUNK, D), dt),
            pltpu.SemaphoreType.DMA,
            pltpu.SemaphoreType.DMA,
            pltpu.SemaphoreType.DMA,
            pltpu.SemaphoreType.DMA,
            pltpu.SemaphoreType.DMA,
            pltpu.SemaphoreType.DMA,
            pltpu.SemaphoreType.DMA,
            pltpu.SemaphoreType.DMA,
        ],
    )
    def gather(t0, t1, i0, i1, o0, o1, i0_v, i1_v, b0, b1,
               g0a, g1a, g0b, g1b, w0a, w1a, w0b, w1b):
        c = lax.axis_index("c")
        t = lax.axis_index("s")
        is_fast = c == FAST
        coff = jnp.where(is_fast, t * CPW_F, NS * CPW_F + t * CPW_S)
        npair = jnp.where(is_fast, CPW_F // 2, CPW_S // 2)

        @pl.when(is_fast)
        def _():
            pltpu.sync_copy(i0.at[pl.ds(t * CPW_F, CPW_F)], i0_v)
            pltpu.sync_copy(i1.at[pl.ds(t * CPW_F, CPW_F)], i1_v)

        @pl.when(jnp.logical_not(is_fast))
        def _():
            s0 = NS * CPW_F + t * CPW_S
            pltpu.sync_copy(i0.at[pl.ds(s0, CPW_S)], i0_v.at[pl.ds(0, CPW_S)])
            pltpu.sync_copy(i1.at[pl.ds(s0, CPW_S)], i1_v.at[pl.ds(0, CPW_S)])

        base = coff * CHUNK
        gsem = ((g0a, g1a), (g0b, g1b))
        wsem = ((w0a, w1a), (w0b, w1b))
        bufs = ((b0.at[0], b1.at[0]), (b0.at[1], b1.at[1]))

        HC = CHUNK // 2

        def fire_g(j, k):
            pltpu.async_copy(
                t0.at[i0_v.at[j, pl.ds(0, HC)]], bufs[k][0].at[pl.ds(0, HC)], gsem[k][0])
            pltpu.async_copy(
                t0.at[i0_v.at[j, pl.ds(HC, HC)]], bufs[k][0].at[pl.ds(HC, HC)], gsem[k][0])
            pltpu.async_copy(
                t1.at[i1_v.at[j, pl.ds(0, HC)]], bufs[k][1].at[pl.ds(0, HC)], gsem[k][1])
            pltpu.async_copy(
                t1.at[i1_v.at[j, pl.ds(HC, HC)]], bufs[k][1].at[pl.ds(HC, HC)], gsem[k][1])

        def wait_g(j, k):
            pltpu.make_async_copy(t0.at[i0_v.at[j]], bufs[k][0], gsem[k][0]).wait()
            pltpu.make_async_copy(t1.at[i1_v.at[j]], bufs[k][1], gsem[k][1]).wait()

        def fire_w(j, k):
            d = pl.ds(base + j * CHUNK, CHUNK)
            pltpu.async_copy(bufs[k][0], o0.at[d], wsem[k][0])
            pltpu.async_copy(bufs[k][1], o1.at[d], wsem[k][1])

        def wait_w(j, k):
            d = pl.ds(base + j * CHUNK, CHUNK)
            pltpu.make_async_copy(bufs[k][0], o0.at[d], wsem[k][0]).wait()
            pltpu.make_async_copy(bufs[k][1], o1.at[d], wsem[k][1]).wait()

        fire_g(0, 0)

        def body(jj, carry):
            j0 = 2 * jj
            j1 = j0 + 1

            @pl.when(jj > 0)
            def _():
                wait_w(j1 - 2, 1)

            fire_g(j1, 1)
            wait_g(j0, 0)
            fire_w(j0, 0)
            wait_g(j1, 1)
            wait_w(j0, 0)

            @pl.when(jj + 1 < npair)
            def _():
                fire_g(j0 + 2, 0)

            fire_w(j1, 1)
            return carry

        lax.fori_loop(0, npair, body, 0)
        wait_w(2 * npair - 1, 1)

    return gather


def _make_scatter():
    @functools.partial(
        pl.kernel,
        out_type=(
            jax.ShapeDtypeStruct((N_PAD, HID), f32),
            jax.ShapeDtypeStruct((N_PAD, HID), f32),
        ),
        mesh=_mesh(),
        scratch_types=[
            pltpu.VMEM((CPW, CHUNK), i32),
            pltpu.VMEM((2, CHUNK, HID), f32),
            pltpu.VMEM_SHARED((N_PAD, HID), f32),
            pltpu.SemaphoreType.DMA,
            pltpu.SemaphoreType.DMA,
        ],
    )
    def scatter(m, idx, zeros, o0, o1, i_v, buf, acc, la, lb):
        c = lax.axis_index("c")
        t = lax.axis_index("s")
        wid = t * NC + c
        pltpu.sync_copy(zeros, buf.at[0])

        def zbody(k, carry):
            pltpu.sync_copy(buf.at[0], acc.at[pl.ds(t * RPT + k * CHUNK, CHUNK)])
            return carry

        lax.fori_loop(0, RPT // CHUNK, zbody, 0)
        plsc.subcore_barrier()

        pltpu.sync_copy(idx.at[wid], i_v)
        base = wid * EPW
        sems = (la, lb)

        def fire_l(j, k):
            pltpu.async_copy(m.at[pl.ds(base + j * CHUNK, CHUNK)], buf.at[k], sems[k])

        def wait_l(j, k):
            pltpu.make_async_copy(
                m.at[pl.ds(base + j * CHUNK, CHUNK)], buf.at[k], sems[k]).wait()

        fire_l(0, 0)

        def body(jj, carry):
            j0 = 2 * jj
            j1 = j0 + 1
            fire_l(j1, 1)
            wait_l(j0, 0)
            pltpu.sync_copy(buf.at[0], acc.at[i_v.at[j0]], add=True)

            @pl.when(jj + 1 < CPW // 2)
            def _():
                fire_l(j0 + 2, 0)

            wait_l(j1, 1)
            pltpu.sync_copy(buf.at[1], acc.at[i_v.at[j1]], add=True)
            return carry

        lax.fori_loop(0, CPW // 2, body, 0)
        plsc.subcore_barrier()

        def wbody(k, carry):
            r = t * RPT + k * CHUNK
            pltpu.sync_copy(acc.at[pl.ds(r, CHUNK)], buf.at[0])

            @pl.when(c == 0)
            def _():
                pltpu.sync_copy(buf.at[0], o0.at[pl.ds(r, CHUNK)])

            @pl.when(c == 1)
            def _():
                pltpu.sync_copy(buf.at[0], o1.at[pl.ds(r, CHUNK)])

            return carry

        lax.fori_loop(0, RPT // CHUNK, wbody, 0)

    return scatter


def _make_counts():
    @functools.partial(
        pl.kernel,
        out_type=(
            jax.ShapeDtypeStruct((N_PAD, CW), f32),
            jax.ShapeDtypeStruct((N_PAD, CW), f32),
        ),
        compiler_params=pltpu.CompilerParams(use_tc_tiling_on_sc=False),
        mesh=_mesh(),
        scratch_types=[
            pltpu.VMEM((CPW, CHUNK), i32),
            pltpu.VMEM((CHUNK, CW), f32),
            pltpu.VMEM((CHUNK, CW), f32),
            pltpu.VMEM_SHARED((N_PAD, CW), f32),
            pltpu.SemaphoreType.DMA,
        ],
    )
    def counts(idx, zeros, ones, o0, o1, i_v, zbuf, obuf, acc, sem):
        c = lax.axis_index("c")
        t = lax.axis_index("s")
        wid = t * NC + c
        pltpu.sync_copy(zeros, zbuf)
        pltpu.sync_copy(ones, obuf)

        def zbody(k, carry):
            pltpu.sync_copy(zbuf, acc.at[pl.ds(t * RPT + k * CHUNK, CHUNK)])
            return carry

        lax.fori_loop(0, RPT // CHUNK, zbody, 0)
        plsc.subcore_barrier()

        pltpu.sync_copy(idx.at[wid], i_v)

        def body(j, carry):
            pltpu.sync_copy(obuf, acc.at[i_v.at[j]], add=True)
            return carry

        lax.fori_loop(0, CPW, body, 0)
        plsc.subcore_barrier()

        def wbody(k, carry):
            r = t * RPT + k * CHUNK
            pltpu.sync_copy(acc.at[pl.ds(r, CHUNK)], zbuf)

            @pl.when(c == 0)
            def _():
                pltpu.sync_copy(zbuf, o0.at[pl.ds(r, CHUNK)])

            @pl.when(c == 1)
            def _():
                pltpu.sync_copy(zbuf, o1.at[pl.ds(r, CHUNK)])

            return carry

        lax.fori_loop(0, RPT // CHUNK, wbody, 0)

    return counts


_BR_E = 1280
_BR_N = 1000


def _premul(x, wa, wb, dout):
    def body(x_r, wa_r, wb_r, a_r, b_r):
        xv = x_r[:]
        a_r[:] = _pack(jnp.dot(xv, wa_r[:], preferred_element_type=f32), dout // 2)
        b_r[:] = _pack(jnp.dot(xv, wb_r[:], preferred_element_type=f32), dout // 2)

    return pl.pallas_call(
        body,
        grid=(N // _BR_N,),
        in_specs=[
            pl.BlockSpec((_BR_N, HID), lambda i: (i, 0)),
            pl.BlockSpec((HID, dout), lambda i: (0, 0)),
            pl.BlockSpec((HID, dout), lambda i: (0, 0)),
        ],
        out_specs=[
            pl.BlockSpec((_BR_N, dout // 2), lambda i: (i, 0)),
            pl.BlockSpec((_BR_N, dout // 2), lambda i: (i, 0)),
        ],
        out_shape=[
            jax.ShapeDtypeStruct((N, dout // 2), i32),
            jax.ShapeDtypeStruct((N, dout // 2), i32),
        ],
    )(x, wa, wb)


def _edge_mlp(g0, g1, ea, wc, b1):
    nea = ea.shape[0] // _BR_E - 1

    def body(g0_r, g1_r, e_r, c_r, b1_r, o_r):
        z = (_unpack(g0_r[:], HID // 2) + _unpack(g1_r[:], HID // 2)
             + jnp.dot(e_r[:], c_r[:], preferred_element_type=f32) + b1_r[:])
        o_r[:] = jnp.maximum(z, 0.0)

    return pl.pallas_call(
        body,
        grid=(E_PAD // _BR_E,),
        in_specs=[
            pl.BlockSpec((_BR_E, HID // 2), lambda i: (i, 0)),
            pl.BlockSpec((_BR_E, HID // 2), lambda i: (i, 0)),
            pl.BlockSpec((_BR_E, DE), lambda i: (jnp.minimum(i, nea), 0)),
            pl.BlockSpec((DE, HID), lambda i: (0, 0)),
            pl.BlockSpec((1, HID), lambda i: (0, 0)),
        ],
        out_specs=pl.BlockSpec((_BR_E, HID), lambda i: (i, 0)),
        out_shape=jax.ShapeDtypeStruct((E_PAD, HID), f32),
    )(g0, g1, ea, wc, b1)


def _node_mlp(x, s0, s1, c0, c1, mw2, mb2, wx, wm, b1, w2, b2, wa, wb, dout):
    def body(x_r, s0_r, s1_r, c0_r, c1_r, mw2_r, mb2_r, wx_r, wm_r, b1_r, w2_r, b2_r,
             wa_r, wb_r, h_r, a_r, b_r):
        craw = c0_r[:, 0:1] + c1_r[:, 0:1]
        cnt = jnp.maximum(craw, 1.0)
        rz = (s0_r[:] + s1_r[:]) / cnt
        mean = (jnp.dot(rz, mw2_r[:], preferred_element_type=f32) + mb2_r[:]
                ) * (craw > 0.0)
        u = jnp.maximum(
            jnp.dot(x_r[:], wx_r[:], preferred_element_type=f32)
            + jnp.dot(mean, wm_r[:], preferred_element_type=f32) + b1_r[:], 0.0)
        h = jnp.maximum(
            jnp.dot(u, w2_r[:], preferred_element_type=f32) + b2_r[:], 0.0)
        h_r[:] = h
        a_r[:] = _pack(jnp.dot(h, wa_r[:], preferred_element_type=f32), dout // 2)
        b_r[:] = _pack(jnp.dot(h, wb_r[:], preferred_element_type=f32), dout // 2)

    return pl.pallas_call(
        body,
        grid=(N // _BR_N,),
        in_specs=[
            pl.BlockSpec((_BR_N, HID), lambda i: (i, 0)),
            pl.BlockSpec((_BR_N, HID), lambda i: (i, 0)),
            pl.BlockSpec((_BR_N, HID), lambda i: (i, 0)),
            pl.BlockSpec((_BR_N, CW), lambda i: (i, 0)),
            pl.BlockSpec((_BR_N, CW), lambda i: (i, 0)),
            pl.BlockSpec((HID, HID), lambda i: (0, 0)),
            pl.BlockSpec((1, HID), lambda i: (0, 0)),
            pl.BlockSpec((HID, HID), lambda i: (0, 0)),
            pl.BlockSpec((HID, HID), lambda i: (0, 0)),
            pl.BlockSpec((1, HID), lambda i: (0, 0)),
            pl.BlockSpec((HID, HID), lambda i: (0, 0)),
            pl.BlockSpec((1, HID), lambda i: (0, 0)),
            pl.BlockSpec((HID, dout), lambda i: (0, 0)),
            pl.BlockSpec((HID, dout), lambda i: (0, 0)),
        ],
        out_specs=[
            pl.BlockSpec((_BR_N, HID), lambda i: (i, 0)),
            pl.BlockSpec((_BR_N, dout // 2), lambda i: (i, 0)),
            pl.BlockSpec((_BR_N, dout // 2), lambda i: (i, 0)),
        ],
        out_shape=[
            jax.ShapeDtypeStruct((N, HID), f32),
            jax.ShapeDtypeStruct((N, dout // 2), i32),
            jax.ShapeDtypeStruct((N, dout // 2), i32),
        ],
    )(x, s0, s1, c0, c1, mw2, mb2, wx, wm, b1, w2, b2, wa, wb)


def _decoder_mlp(g0, g1, ea, wc, b1, w2, b2, w3, b3):
    nea = ea.shape[0] // _BR_E - 1

    def body(g0_r, g1_r, e_r, c_r, b1_r, w2_r, b2_r, w3_r, b3_r, o_r):
        z = jnp.maximum(
            _unpack(g0_r[:], DEC // 2) + _unpack(g1_r[:], DEC // 2)
            + jnp.dot(e_r[:], c_r[:], preferred_element_type=f32) + b1_r[:], 0.0)
        d = jnp.maximum(jnp.dot(z, w2_r[:], preferred_element_type=f32) + b2_r[:], 0.0)
        o_r[:] = jnp.dot(d, w3_r[:], preferred_element_type=f32) + b3_r[0, 0]

    return pl.pallas_call(
        body,
        grid=(E_PAD // _BR_E,),
        in_specs=[
            pl.BlockSpec((_BR_E, DEC // 2), lambda i: (i, 0)),
            pl.BlockSpec((_BR_E, DEC // 2), lambda i: (i, 0)),
            pl.BlockSpec((_BR_E, DE), lambda i: (jnp.minimum(i, nea), 0)),
            pl.BlockSpec((DE, DEC), lambda i: (0, 0)),
            pl.BlockSpec((1, DEC), lambda i: (0, 0)),
            pl.BlockSpec((DEC, DEC // 2), lambda i: (0, 0)),
            pl.BlockSpec((1, DEC // 2), lambda i: (0, 0)),
            pl.BlockSpec((DEC // 2, 1), lambda i: (0, 0)),
            pl.BlockSpec((1, 1), lambda i: (0, 0)),
        ],
        out_specs=pl.BlockSpec((_BR_E, 1), lambda i: (i, 0)),
        out_shape=jax.ShapeDtypeStruct((E_PAD, 1), f32),
    )(g0, g1, ea, wc, b1, w2, b2, w3, b3)


def kernel(x, edge_index, edge_attr, params):
    src = edge_index[0]
    dst = edge_index[1]
    pad0 = jnp.zeros((E_PAD - E,), i32)
    padn = jnp.full((E_PAD - E,), N, i32)
    dst_g = jnp.concatenate([dst, pad0]).reshape(TOTCH, CHUNK)
    src_g = jnp.concatenate([src, pad0]).reshape(TOTCH, CHUNK)
    dst_s = jnp.concatenate([dst, padn]).reshape(NW, CPW, CHUNK)
    zeros_h = jnp.zeros((CHUNK, HID), f32)
    zeros_c = jnp.zeros((CHUNK, CW), f32)
    ones_c = jnp.ones((CHUNK, CW), f32)

    gather_h = _make_gather(HID // 2)
    gather_d = _make_gather(DEC // 2)
    scatter = _make_scatter()
    counts = _make_counts()

    c0, c1 = counts(dst_s, zeros_c, ones_c)

    w1 = params['l0_msg_W1']
    xa, xb = _premul(x, w1[:HID], w1[HID:2 * HID], HID)
    h = x
    for l in range(3):
        w1 = params['l%d_msg_W1' % l]
        g0, g1 = gather_h(xa, xb, dst_g, src_g)
        rz = _edge_mlp(
            g0, g1, edge_attr, w1[2 * HID:],
            params['l%d_msg_b1' % l].reshape(1, HID))
        s0, s1 = scatter(rz, dst_s, zeros_h)
        nw1 = params['l%d_node_W1' % l]
        if l < 2:
            nxt = params['l%d_msg_W1' % (l + 1)]
            wa, wb, dout = nxt[:HID], nxt[HID:2 * HID], HID
        else:
            dw1 = params['dec_W1']
            wa, wb, dout = dw1[:HID], dw1[HID:2 * HID], DEC
        h, xa, xb = _node_mlp(
            h, s0, s1, c0, c1,
            params['l%d_msg_W2' % l],
            params['l%d_msg_b2' % l].reshape(1, HID),
            nw1[:HID], nw1[HID:],
            params['l%d_node_b1' % l].reshape(1, HID),
            params['l%d_node_W2' % l],
            params['l%d_node_b2' % l].reshape(1, HID),
            wa, wb, dout)

    gd0, gd1 = gather_d(xa, xb, src_g, dst_g)
    out = _decoder_mlp(
        gd0, gd1, edge_attr, params['dec_W1'][2 * HID:],
        params['dec_b1'].reshape(1, DEC),
        params['dec_W2'],
        params['dec_b2'].reshape(1, DEC // 2),
        params['dec_W3'],
        params['dec_b3'].reshape(1, 1))
    return out[:E, 0]
